# Initial kernel scaffold; baseline (speedup 1.0000x reference)
import jax
import jax.numpy as jnp
from jax.experimental import pallas as pl


def kernel(x, edge_index, Wl1, bl1, Wr1, Wl2, bl2, Wr2, Wf, bf):
    raise NotImplementedError("write your pallas kernel here")


# speedup vs baseline: 6.7807x; 6.7807x over previous
import functools

import jax
import jax.numpy as jnp
from jax import lax
from jax.experimental import pallas as pl
from jax.experimental.pallas import tpu as pltpu
from jax.experimental.pallas import tpu_sc as plsc

N = 10000
E = 320000
D = 128
LABELS = 64

NC = 2
NS = 16
NW = NC * NS
EPW = E // NW
C = 125
NCH = EPW // C
RPT = N // NS
RCH = RPT // C
DW = 128
WRT = 640
WLAST_BASE = WRT * (NS - 1)
WLAST = N - WLAST_BASE


def _make_sc_deg():
    def body(dst_hbm, deg_out, dst_v, ones_v, deg_sh, sem):
        cid = lax.axis_index("c")
        sid = lax.axis_index("s")
        wid = cid * NS + sid
        pltpu.sync_copy(dst_hbm.at[wid], dst_v)

        zv = jnp.zeros((16,), jnp.float32)

        def zrow(r, _):
            for col in range(DW // 16):
                ones_v[r, pl.ds(col * 16, 16)] = zv
            return 0

        lax.fori_loop(0, C, zrow, 0)
        base = sid * RPT
        for i in range(RCH):
            pltpu.sync_copy(ones_v, deg_sh.at[pl.ds(base + i * C, C)])

        ov = jnp.ones((16,), jnp.float32)

        def orow(r, _):
            for col in range(DW // 16):
                ones_v[r, pl.ds(col * 16, 16)] = ov
            return 0

        lax.fori_loop(0, C, orow, 0)
        plsc.subcore_barrier()

        def chunk(j, _):
            pltpu.sync_copy(ones_v, deg_sh.at[dst_v.at[j]], add=True)
            return 0

        lax.fori_loop(0, NCH, chunk, 0)
        plsc.subcore_barrier()

        wbase = sid * WRT

        @pl.when(sid < NS - 1)
        def _():
            pltpu.sync_copy(deg_sh.at[pl.ds(wbase, WRT)],
                            deg_out.at[cid, pl.ds(wbase, WRT)])

        @pl.when(sid == NS - 1)
        def _():
            pltpu.sync_copy(deg_sh.at[pl.ds(WLAST_BASE, WLAST)],
                            deg_out.at[cid, pl.ds(WLAST_BASE, WLAST)])

    mesh = plsc.VectorSubcoreMesh(core_axis_name="c", subcore_axis_name="s", num_cores=NC, num_subcores=NS)
    return pl.kernel(
        body,
        out_type=jax.ShapeDtypeStruct((NC, N, DW), jnp.float32),
        mesh=mesh,
        scratch_types=[
            pltpu.VMEM((NCH, C), jnp.int32),
            pltpu.VMEM((C, DW), jnp.float32),
            pltpu.VMEM_SHARED((N, DW), jnp.float32),
            pltpu.SemaphoreType.DMA,
        ],
        name="sc_deg")


def _make_sc_agg(compute_deg: bool):
    out_type = [jax.ShapeDtypeStruct((NC, N, D), jnp.float32)]
    scratch = [
        pltpu.VMEM((NCH, C), jnp.int32),
        pltpu.VMEM((NCH, C), jnp.int32),
        pltpu.VMEM((C, D), jnp.float32),
        pltpu.VMEM_SHARED((N, D), jnp.float32),
        pltpu.SemaphoreType.DMA,
    ]
    if compute_deg:
        out_type.append(jax.ShapeDtypeStruct((NC, N, DW), jnp.float32))
        scratch.insert(3, pltpu.VMEM((C, DW), jnp.float32))
        scratch.insert(5, pltpu.VMEM_SHARED((N, DW), jnp.float32))

    def body(y_hbm, src_hbm, dst_hbm, *rest):
        if compute_deg:
            (agg_out, deg_out, src_v, dst_v, stage, ones_v,
             agg_sh, deg_sh, sem) = rest
        else:
            agg_out, src_v, dst_v, stage, agg_sh, sem = rest
        cid = lax.axis_index("c")
        sid = lax.axis_index("s")
        wid = cid * NS + sid

        pltpu.sync_copy(src_hbm.at[wid], src_v)
        pltpu.sync_copy(dst_hbm.at[wid], dst_v)

        zv = jnp.zeros((16,), jnp.float32)

        def zrow(r, _):
            for col in range(D // 16):
                stage[r, pl.ds(col * 16, 16)] = zv
            if compute_deg:
                ones_v[r, :] = zv
            return 0

        lax.fori_loop(0, C, zrow, 0)

        base = sid * RPT
        for i in range(RCH):
            pltpu.sync_copy(stage, agg_sh.at[pl.ds(base + i * C, C)])
            if compute_deg:
                pltpu.sync_copy(ones_v, deg_sh.at[pl.ds(base + i * C, C)])

        if compute_deg:
            ov = jnp.ones((16,), jnp.float32)

            def orow(r, _):
                ones_v[r, :] = ov
                return 0

            lax.fori_loop(0, C, orow, 0)

        plsc.subcore_barrier()

        def chunk(j, _):
            pltpu.async_copy(y_hbm.at[src_v.at[j]], stage, sem).wait()
            pltpu.sync_copy(stage, agg_sh.at[dst_v.at[j]], add=True)
            if compute_deg:
                pltpu.sync_copy(ones_v, deg_sh.at[dst_v.at[j]], add=True)
            return 0

        lax.fori_loop(0, NCH, chunk, 0)

        plsc.subcore_barrier()

        wbase = sid * WRT

        @pl.when(sid < NS - 1)
        def _():
            pltpu.sync_copy(agg_sh.at[pl.ds(wbase, WRT)],
                            agg_out.at[cid, pl.ds(wbase, WRT)])
            if compute_deg:
                pltpu.sync_copy(deg_sh.at[pl.ds(wbase, WRT)],
                                deg_out.at[cid, pl.ds(wbase, WRT)])

        @pl.when(sid == NS - 1)
        def _():
            pltpu.sync_copy(agg_sh.at[pl.ds(WLAST_BASE, WLAST)],
                            agg_out.at[cid, pl.ds(WLAST_BASE, WLAST)])
            if compute_deg:
                pltpu.sync_copy(deg_sh.at[pl.ds(WLAST_BASE, WLAST)],
                                deg_out.at[cid, pl.ds(WLAST_BASE, WLAST)])

    mesh = plsc.VectorSubcoreMesh(core_axis_name="c", subcore_axis_name="s", num_cores=NC, num_subcores=NS)
    return pl.kernel(body, out_type=tuple(out_type), mesh=mesh,
                     scratch_types=scratch,
                     name="sc_seg_sum_deg" if compute_deg else "sc_seg_sum")


_sc_deg = _make_sc_deg()
_sc_agg = _make_sc_agg(False)

_PREC = lax.Precision.HIGHEST


def _pre_body(x_ref, wl_ref, y_ref):
    y_ref[...] = jnp.dot(x_ref[...], wl_ref[...].T,
                         preferred_element_type=jnp.float32, precision=_PREC)


def _elu(z):
    return jnp.where(z > 0, z, jnp.exp(jnp.minimum(z, 0.0)) - 1.0)


def _mid_body(agg_ref, deg_ref, x_ref, wr_ref, bl_ref, wln_ref, h_ref, y2_ref):
    deg = deg_ref[0, :, 0:1] + deg_ref[1, :, 0:1]
    rdeg = 1.0 / jnp.maximum(deg, 1.0)
    mean = (agg_ref[0] + agg_ref[1]) * rdeg
    root = jnp.dot(x_ref[...], wr_ref[...].T,
                   preferred_element_type=jnp.float32, precision=_PREC)
    h = _elu(mean + bl_ref[...] + root)
    h_ref[...] = h
    y2_ref[...] = jnp.dot(h, wln_ref[...].T,
                          preferred_element_type=jnp.float32, precision=_PREC)


def _fin_body(agg_ref, deg_ref, h_ref, wr_ref, bl_ref, wf_ref, bf_ref, o_ref):
    deg = deg_ref[0, :, 0:1] + deg_ref[1, :, 0:1]
    rdeg = 1.0 / jnp.maximum(deg, 1.0)
    mean = (agg_ref[0] + agg_ref[1]) * rdeg
    root = jnp.dot(h_ref[...], wr_ref[...].T,
                   preferred_element_type=jnp.float32, precision=_PREC)
    h2 = _elu(mean + bl_ref[...] + root)
    logits = jnp.dot(h2, wf_ref[...].T,
                     preferred_element_type=jnp.float32, precision=_PREC)
    o_ref[...] = jax.nn.sigmoid(logits + bf_ref[...])


def kernel(x, edge_index, Wl1, bl1, Wr1, Wl2, bl2, Wr2, Wf, bf):
    src = edge_index[0].astype(jnp.int32).reshape(NW, NCH, C)
    dst = edge_index[1].astype(jnp.int32).reshape(NW, NCH, C)
    bl1r = bl1.reshape(1, D)
    bl2r = bl2.reshape(1, D)
    bfr = bf.reshape(1, LABELS)

    y1 = pl.pallas_call(
        _pre_body,
        out_shape=jax.ShapeDtypeStruct((N, D), jnp.float32),
    )(x, Wl1)

    deg = _sc_deg(dst)
    (agg1,) = _sc_agg(y1, src, dst)

    h, y2 = pl.pallas_call(
        _mid_body,
        out_shape=[jax.ShapeDtypeStruct((N, D), jnp.float32),
                   jax.ShapeDtypeStruct((N, D), jnp.float32)],
    )(agg1, deg, x, Wr1, bl1r, Wl2)

    (agg2,) = _sc_agg(y2, src, dst)

    out = pl.pallas_call(
        _fin_body,
        out_shape=jax.ShapeDtypeStruct((N, LABELS), jnp.float32),
    )(agg2, deg, h, Wr2, bl2r, Wf, bfr)
    return out
